# baseline (speedup 1.0000x reference)
import functools

import jax
import jax.numpy as jnp
from jax import lax
from jax.experimental import pallas as pl
from jax.experimental.pallas import tpu as pltpu
from jax.experimental.pallas import tpu_sc as plsc

_NC = 2
_NS = 16
_NW = _NC * _NS
_L = 16


def _rot_perm(lane, r):
  return (lane + r) & (_L - 1)


def _gather_lanes(x, perm):
  return jnp.take_along_axis(x, perm, axis=0,
                             mode=lax.GatherScatterMode.PROMISE_IN_BOUNDS)


_CBLK = 16


def _sc_body_copy(m, b, d):
  span = -(-m // _NW)
  span = -(-span // 8) * 8
  nblk = -(-span // _CBLK)
  mesh = plsc.VectorSubcoreMesh(
      core_axis_name="c", subcore_axis_name="s", num_cores=_NC,
      num_subcores=_NS)

  @functools.partial(
      pl.kernel,
      out_type=(jax.ShapeDtypeStruct((m + b, d), jnp.float32),
                jax.ShapeDtypeStruct((m + b,), jnp.int32),
                jax.ShapeDtypeStruct((m + b,), jnp.int32)),
      mesh=mesh,
      scratch_types=[
          pltpu.VMEM((_CBLK, d), jnp.float32),
          pltpu.VMEM((_CBLK, d), jnp.float32),
          pltpu.VMEM((span,), jnp.int32),
          pltpu.SemaphoreType.DMA,
          pltpu.SemaphoreType.DMA,
          pltpu.SemaphoreType.DMA,
          pltpu.SemaphoreType.DMA,
          pltpu.SemaphoreType.DMA,
      ],
  )
  def sc(bx_hbm, by_hbm, bt_hbm, ox_ref, oy_ref, ot_ref,
         buf_a, buf_b, ibuf, rs_a, rs_b, ws_a, ws_b, isem):
    cid = lax.axis_index("c")
    sid = lax.axis_index("s")
    tid = cid * _NS + sid
    lo = pl.multiple_of(jnp.minimum(tid * span, m - span), 8)

    for src, dst in ((by_hbm, oy_ref), (bt_hbm, ot_ref)):
      pltpu.async_copy(src.at[pl.ds(lo, span)], ibuf, isem).wait()
      pltpu.async_copy(ibuf, dst.at[pl.ds(lo, span)], isem).wait()

    bufs = (buf_a, buf_b)
    rsems = (rs_a, rs_b)
    wsems = (ws_a, ws_b)
    starts = []
    for j in range(nblk):
      starts.append(
          pl.multiple_of(lo + jnp.minimum(j * _CBLK, span - _CBLK), 8))
    for j in range(nblk):
      k = j % 2
      if j >= 2:
        pltpu.make_async_copy(bufs[k], ox_ref.at[pl.ds(starts[j - 2], _CBLK)],
                              wsems[k]).wait()
      pltpu.make_async_copy(bx_hbm.at[pl.ds(starts[j], _CBLK)], bufs[k],
                            rsems[k]).start()
      pltpu.make_async_copy(bx_hbm.at[pl.ds(starts[j], _CBLK)], bufs[k],
                            rsems[k]).wait()
      pltpu.make_async_copy(bufs[k], ox_ref.at[pl.ds(starts[j], _CBLK)],
                            wsems[k]).start()
    for j in range(max(nblk - 2, 0), nblk):
      k = j % 2
      pltpu.make_async_copy(bufs[k], ox_ref.at[pl.ds(starts[j], _CBLK)],
                            wsems[k]).wait()

  return sc


def _sc_sparse(m, b, d, ox_ref, oy_ref, ot_ref):
  nchunk = b // _L
  ipt = b // _NW
  cpt = ipt // _L
  mesh = plsc.VectorSubcoreMesh(
      core_axis_name="c", subcore_axis_name="s", num_cores=_NC,
      num_subcores=_NS)

  @functools.partial(
      pl.kernel,
      out_type=(),
      mesh=mesh,
      scratch_types=[
          pltpu.VMEM((b,), jnp.int32),
          pltpu.VMEM((cpt, _L), jnp.int32),
          pltpu.VMEM((ipt,), jnp.int32),
          pltpu.VMEM((ipt,), jnp.int32),
          pltpu.VMEM((_L, d), jnp.float32),
          pltpu.VMEM((ipt,), jnp.int32),
          pltpu.VMEM((ipt,), jnp.int32),
          pltpu.SemaphoreType.DMA,
      ],
  )
  def sc(bx_hbm, inx_hbm, by_hbm, bt_hbm, iny_hbm, idx_hbm, idx3_hbm,
         itv_hbm,
         idx_v, myidx_v, myflat_v, w_v, rows_v, val_v, tail_v, sem):
    cid = lax.axis_index("c")
    sid = lax.axis_index("s")
    tid = cid * _NS + sid
    base = tid * ipt
    lane = lax.iota(jnp.int32, _L)

    pltpu.sync_copy(idx_hbm, idx_v)
    pltpu.sync_copy(idx_hbm.at[pl.ds(base, ipt)], myflat_v)
    pltpu.sync_copy(idx3_hbm.at[tid], myidx_v)

    for j in range(cpt):
      g = tid * cpt + j
      v = idx_v[pl.ds(g * _L, _L)]
      best = g * _L + lane

      def wbody(c, best, v=v):
        u = idx_v[pl.ds(c * _L, _L)]
        for r in range(_L):
          perm = _rot_perm(lane, r)
          ur = u if r == 0 else _gather_lanes(u, perm)
          jr = c * _L + perm
          upd = jnp.logical_and(ur == v, jr > best)
          best = jnp.where(upd, jr, best)
        return best

      w_v[pl.ds(j * _L, _L)] = lax.fori_loop(0, nchunk, wbody, best)

    for cc in range(cpt):
      pltpu.async_copy(inx_hbm.at[w_v.at[pl.ds(cc * _L, _L)]],
                       rows_v, sem).wait()
      pltpu.async_copy(rows_v, ox_ref.at[myidx_v.at[cc]], sem).wait()
      pltpu.async_copy(bx_hbm.at[myidx_v.at[cc]], rows_v, sem).wait()
      pltpu.sync_copy(
          rows_v,
          ox_ref.at[pl.ds(pl.multiple_of(m + base + cc * _L, 8), _L)])

    tail_at = pl.ds(pl.multiple_of(m + base, 8), ipt)
    pltpu.async_copy(by_hbm.at[myflat_v], tail_v, sem).wait()
    pltpu.sync_copy(tail_v, oy_ref.at[tail_at])
    pltpu.async_copy(bt_hbm.at[myflat_v], tail_v, sem).wait()
    pltpu.sync_copy(tail_v, ot_ref.at[tail_at])
    pltpu.async_copy(iny_hbm.at[w_v], val_v, sem).wait()
    pltpu.async_copy(val_v, oy_ref.at[myflat_v], sem).wait()
    pltpu.sync_copy(itv_hbm, val_v)
    pltpu.async_copy(val_v, ot_ref.at[myflat_v], sem).wait()

  return sc


def kernel(bx, by, bt, in_x, in_y, in_t, swap_idx):
  m = bx.shape[0]
  b = in_x.shape[0]
  d = 1
  for s in bx.shape[1:]:
    d *= s
  assert b % (_NW * _L) == 0 and m % 8 == 0

  bx_f = bx.reshape(m, d)
  inx_f = in_x.reshape(b, d)
  idx3 = swap_idx.reshape(_NW, b // _NW // _L, _L)
  itv = jnp.full((b // _NW,), in_t, dtype=jnp.int32)

  body_x, body_y, body_t = _sc_body_copy(m, b, d)(bx_f, by, bt)
  ox_ref = jax.new_ref(body_x)
  oy_ref = jax.new_ref(body_y)
  ot_ref = jax.new_ref(body_t)
  _sc_sparse(m, b, d, ox_ref, oy_ref, ot_ref)(
      bx_f, inx_f, by, bt, in_y, swap_idx, idx3, itv)
  out_bx = ox_ref[...].reshape((m + b,) + bx.shape[1:])
  return (out_bx, oy_ref[...], ot_ref[...])


# speedup vs baseline: 1.0273x; 1.0273x over previous
import functools

import jax
import jax.numpy as jnp
from jax import lax
from jax.experimental import pallas as pl
from jax.experimental.pallas import tpu as pltpu
from jax.experimental.pallas import tpu_sc as plsc

_NC = 2
_NS = 16
_NW = _NC * _NS
_L = 16
_BLK = 1000


def _rot_perm(lane, r):
  return (lane + r) & (_L - 1)


def _gather_lanes(x, perm):
  return jnp.take_along_axis(x, perm, axis=0,
                             mode=lax.GatherScatterMode.PROMISE_IN_BOUNDS)


def _tc_body_copy(m, mp, b, d):
  def body(bx_ref, by_ref, bt_ref, ox_ref, oy_ref, ot_ref, sem):
    i = pl.program_id(0)
    ox_ref[...] = bx_ref[...]

    @pl.when(i == 0)
    def _ints():
      pltpu.make_async_copy(by_ref, oy_ref.at[pl.ds(0, mp)], sem).start()
      pltpu.make_async_copy(bt_ref, ot_ref.at[pl.ds(0, mp)], sem).start()
      pltpu.make_async_copy(by_ref, oy_ref.at[pl.ds(0, mp)], sem).wait()
      pltpu.make_async_copy(bt_ref, ot_ref.at[pl.ds(0, mp)], sem).wait()

  return pl.pallas_call(
      body,
      grid=(m // _BLK,),
      in_specs=[pl.BlockSpec((_BLK, d), lambda i: (i, 0)),
                pl.BlockSpec(memory_space=pl.ANY),
                pl.BlockSpec(memory_space=pl.ANY)],
      out_specs=[pl.BlockSpec((_BLK, d), lambda i: (i, 0)),
                 pl.BlockSpec(memory_space=pl.ANY),
                 pl.BlockSpec(memory_space=pl.ANY)],
      out_shape=(jax.ShapeDtypeStruct((m + b, d), jnp.float32),
                 jax.ShapeDtypeStruct((m + b,), jnp.int32),
                 jax.ShapeDtypeStruct((m + b,), jnp.int32)),
      scratch_shapes=[pltpu.SemaphoreType.DMA],
  )


_CBLK = 16


def _sc_body_copy(m, b, d):
  span = -(-m // _NW)
  span = -(-span // 8) * 8
  nblk = -(-span // _CBLK)
  mesh = plsc.VectorSubcoreMesh(
      core_axis_name="c", subcore_axis_name="s", num_cores=_NC,
      num_subcores=_NS)

  @functools.partial(
      pl.kernel,
      out_type=(jax.ShapeDtypeStruct((m + b, d), jnp.float32),
                jax.ShapeDtypeStruct((m + b,), jnp.int32),
                jax.ShapeDtypeStruct((m + b,), jnp.int32)),
      mesh=mesh,
      scratch_types=[
          pltpu.VMEM((_CBLK, d), jnp.float32),
          pltpu.VMEM((_CBLK, d), jnp.float32),
          pltpu.VMEM((span,), jnp.int32),
          pltpu.SemaphoreType.DMA,
          pltpu.SemaphoreType.DMA,
          pltpu.SemaphoreType.DMA,
          pltpu.SemaphoreType.DMA,
          pltpu.SemaphoreType.DMA,
      ],
  )
  def sc(bx_hbm, by_hbm, bt_hbm, ox_ref, oy_ref, ot_ref,
         buf_a, buf_b, ibuf, rs_a, rs_b, ws_a, ws_b, isem):
    cid = lax.axis_index("c")
    sid = lax.axis_index("s")
    tid = cid * _NS + sid
    lo = pl.multiple_of(jnp.minimum(tid * span, m - span), 8)

    for src, dst in ((by_hbm, oy_ref), (bt_hbm, ot_ref)):
      pltpu.async_copy(src.at[pl.ds(lo, span)], ibuf, isem).wait()
      pltpu.async_copy(ibuf, dst.at[pl.ds(lo, span)], isem).wait()

    bufs = (buf_a, buf_b)
    rsems = (rs_a, rs_b)
    wsems = (ws_a, ws_b)
    starts = []
    for j in range(nblk):
      starts.append(
          pl.multiple_of(lo + jnp.minimum(j * _CBLK, span - _CBLK), 8))
    for j in range(nblk):
      k = j % 2
      if j >= 2:
        pltpu.make_async_copy(bufs[k], ox_ref.at[pl.ds(starts[j - 2], _CBLK)],
                              wsems[k]).wait()
      pltpu.make_async_copy(bx_hbm.at[pl.ds(starts[j], _CBLK)], bufs[k],
                            rsems[k]).start()
      pltpu.make_async_copy(bx_hbm.at[pl.ds(starts[j], _CBLK)], bufs[k],
                            rsems[k]).wait()
      pltpu.make_async_copy(bufs[k], ox_ref.at[pl.ds(starts[j], _CBLK)],
                            wsems[k]).start()
    for j in range(max(nblk - 2, 0), nblk):
      k = j % 2
      pltpu.make_async_copy(bufs[k], ox_ref.at[pl.ds(starts[j], _CBLK)],
                            wsems[k]).wait()

  return sc


def _sc_sparse(m, b, d, ox_ref, oy_ref, ot_ref):
  nchunk = b // _L
  ipt = b // _NW
  cpt = ipt // _L
  mesh = plsc.VectorSubcoreMesh(
      core_axis_name="c", subcore_axis_name="s", num_cores=_NC,
      num_subcores=_NS)

  assert cpt == 2

  @functools.partial(
      pl.kernel,
      out_type=(),
      mesh=mesh,
      scratch_types=[
          pltpu.VMEM((b,), jnp.int32),
          pltpu.VMEM((cpt, _L), jnp.int32),
          pltpu.VMEM((ipt,), jnp.int32),
          pltpu.VMEM((ipt,), jnp.int32),
          pltpu.VMEM((_L, d), jnp.float32),
          pltpu.VMEM((_L, d), jnp.float32),
          pltpu.VMEM((ipt,), jnp.int32),
          pltpu.VMEM((ipt,), jnp.int32),
          pltpu.VMEM((ipt,), jnp.int32),
      ] + [pltpu.SemaphoreType.DMA] * 9,
  )
  def sc(bx_hbm, inx_hbm, by_hbm, bt_hbm, iny_hbm, idx_hbm, idx3_hbm,
         itv_hbm,
         idx_v, myidx_v, myflat_v, w_v, rows_a, rows_b, val_v, taily_v,
         tailt_v, s0, s1, s2, s3, s4, s5, s6, s7, s8):
    cid = lax.axis_index("c")
    sid = lax.axis_index("s")
    tid = cid * _NS + sid
    base = tid * ipt
    lane = lax.iota(jnp.int32, _L)
    rows = (rows_a, rows_b)

    pltpu.sync_copy(idx_hbm, idx_v)
    pltpu.sync_copy(idx_hbm.at[pl.ds(base, ipt)], myflat_v)
    pltpu.sync_copy(idx3_hbm.at[tid], myidx_v)

    tg = [pltpu.async_copy(bx_hbm.at[myidx_v.at[cc]], rows[cc], (s0, s1)[cc])
          for cc in range(cpt)]
    ty = pltpu.async_copy(by_hbm.at[myflat_v], taily_v, s2)
    tt = pltpu.async_copy(bt_hbm.at[myflat_v], tailt_v, s3)

    for j in range(cpt):
      g = tid * cpt + j
      v = idx_v[pl.ds(g * _L, _L)]
      best = g * _L + lane

      def wbody(c, best, v=v):
        u = idx_v[pl.ds(c * _L, _L)]
        for r in range(_L):
          perm = _rot_perm(lane, r)
          ur = u if r == 0 else _gather_lanes(u, perm)
          jr = c * _L + perm
          upd = jnp.logical_and(ur == v, jr > best)
          best = jnp.where(upd, jr, best)
        return best

      w_v[pl.ds(j * _L, _L)] = lax.fori_loop(0, nchunk, wbody, best)

    vg = pltpu.async_copy(iny_hbm.at[w_v], val_v, s8)

    tail_at = pl.ds(pl.multiple_of(m + base, 8), ipt)
    tw = []
    for cc in range(cpt):
      tg[cc].wait()
      tw.append(pltpu.make_async_copy(
          rows[cc],
          ox_ref.at[pl.ds(pl.multiple_of(m + base + cc * _L, 8), _L)],
          (s4, s5)[cc]))
      tw[cc].start()
    ty.wait()
    tyw = pltpu.make_async_copy(taily_v, oy_ref.at[tail_at], s6)
    tyw.start()
    tt.wait()
    ttw = pltpu.make_async_copy(tailt_v, ot_ref.at[tail_at], s7)
    ttw.start()

    vg.wait()
    yw = pltpu.async_copy(val_v, oy_ref.at[myflat_v], s2)
    pltpu.sync_copy(itv_hbm, val_v)
    tvw = pltpu.async_copy(val_v, ot_ref.at[myflat_v], s3)

    sg = []
    for cc in range(cpt):
      tw[cc].wait()
      sg.append(pltpu.async_copy(inx_hbm.at[w_v.at[pl.ds(cc * _L, _L)]],
                                 rows[cc], (s0, s1)[cc]))
    sw = []
    for cc in range(cpt):
      sg[cc].wait()
      sw.append(pltpu.async_copy(rows[cc], ox_ref.at[myidx_v.at[cc]],
                                 (s4, s5)[cc]))
    for c in sw:
      c.wait()
    yw.wait()
    tvw.wait()
    tyw.wait()
    ttw.wait()

  return sc


def kernel(bx, by, bt, in_x, in_y, in_t, swap_idx):
  m = bx.shape[0]
  b = in_x.shape[0]
  d = 1
  for s in bx.shape[1:]:
    d *= s
  mp = ((m + 127) // 128) * 128
  assert b % (_NW * _L) == 0 and m % _BLK == 0 and mp <= m + b

  bx_f = bx.reshape(m, d)
  inx_f = in_x.reshape(b, d)
  idx3 = swap_idx.reshape(_NW, b // _NW // _L, _L)
  itv = jnp.full((b // _NW,), in_t, dtype=jnp.int32)
  pad = jnp.zeros((mp - m,), dtype=jnp.int32)
  by_p = jnp.concatenate([by, pad])
  bt_p = jnp.concatenate([bt, pad])

  body_x, body_y, body_t = _tc_body_copy(m, mp, b, d)(bx_f, by_p, bt_p)
  ox_ref = jax.new_ref(body_x)
  oy_ref = jax.new_ref(body_y)
  ot_ref = jax.new_ref(body_t)
  _sc_sparse(m, b, d, ox_ref, oy_ref, ot_ref)(
      bx_f, inx_f, by, bt, in_y, swap_idx, idx3, itv)
  out_bx = ox_ref[...].reshape((m + b,) + bx.shape[1:])
  return (out_bx, oy_ref[...], ot_ref[...])


# speedup vs baseline: 1.0291x; 1.0018x over previous
import functools

import jax
import jax.numpy as jnp
from jax import lax
from jax.experimental import pallas as pl
from jax.experimental.pallas import tpu as pltpu
from jax.experimental.pallas import tpu_sc as plsc

_NC = 2
_NS = 16
_NW = _NC * _NS
_L = 16
_BLK = 1000


def _rot_perm(lane, r):
  return (lane + r) & (_L - 1)


def _gather_lanes(x, perm):
  return jnp.take_along_axis(x, perm, axis=0,
                             mode=lax.GatherScatterMode.PROMISE_IN_BOUNDS)


def _tc_body_copy(m, mp, b, d):
  def body(bx_ref, by_ref, bt_ref, ox_ref, oy_ref, ot_ref, sem):
    i = pl.program_id(0)
    ox_ref[...] = bx_ref[...]

    @pl.when(i == 0)
    def _ints():
      pltpu.make_async_copy(by_ref, oy_ref.at[pl.ds(0, mp)], sem).start()
      pltpu.make_async_copy(bt_ref, ot_ref.at[pl.ds(0, mp)], sem).start()
      pltpu.make_async_copy(by_ref, oy_ref.at[pl.ds(0, mp)], sem).wait()
      pltpu.make_async_copy(bt_ref, ot_ref.at[pl.ds(0, mp)], sem).wait()

  return pl.pallas_call(
      body,
      grid=(m // _BLK,),
      in_specs=[pl.BlockSpec((_BLK, d), lambda i: (i, 0)),
                pl.BlockSpec(memory_space=pl.ANY),
                pl.BlockSpec(memory_space=pl.ANY)],
      out_specs=[pl.BlockSpec((_BLK, d), lambda i: (i, 0)),
                 pl.BlockSpec(memory_space=pl.ANY),
                 pl.BlockSpec(memory_space=pl.ANY)],
      out_shape=(jax.ShapeDtypeStruct((m + b, d), jnp.float32),
                 jax.ShapeDtypeStruct((m + b,), jnp.int32),
                 jax.ShapeDtypeStruct((m + b,), jnp.int32)),
      scratch_shapes=[pltpu.SemaphoreType.DMA],
  )


_CBLK = 16


def _sc_body_copy(m, b, d):
  span = -(-m // _NW)
  span = -(-span // 8) * 8
  nblk = -(-span // _CBLK)
  mesh = plsc.VectorSubcoreMesh(
      core_axis_name="c", subcore_axis_name="s", num_cores=_NC,
      num_subcores=_NS)

  @functools.partial(
      pl.kernel,
      out_type=(jax.ShapeDtypeStruct((m + b, d), jnp.float32),
                jax.ShapeDtypeStruct((m + b,), jnp.int32),
                jax.ShapeDtypeStruct((m + b,), jnp.int32)),
      mesh=mesh,
      scratch_types=[
          pltpu.VMEM((_CBLK, d), jnp.float32),
          pltpu.VMEM((_CBLK, d), jnp.float32),
          pltpu.VMEM((span,), jnp.int32),
          pltpu.SemaphoreType.DMA,
          pltpu.SemaphoreType.DMA,
          pltpu.SemaphoreType.DMA,
          pltpu.SemaphoreType.DMA,
          pltpu.SemaphoreType.DMA,
      ],
  )
  def sc(bx_hbm, by_hbm, bt_hbm, ox_ref, oy_ref, ot_ref,
         buf_a, buf_b, ibuf, rs_a, rs_b, ws_a, ws_b, isem):
    cid = lax.axis_index("c")
    sid = lax.axis_index("s")
    tid = cid * _NS + sid
    lo = pl.multiple_of(jnp.minimum(tid * span, m - span), 8)

    for src, dst in ((by_hbm, oy_ref), (bt_hbm, ot_ref)):
      pltpu.async_copy(src.at[pl.ds(lo, span)], ibuf, isem).wait()
      pltpu.async_copy(ibuf, dst.at[pl.ds(lo, span)], isem).wait()

    bufs = (buf_a, buf_b)
    rsems = (rs_a, rs_b)
    wsems = (ws_a, ws_b)
    starts = []
    for j in range(nblk):
      starts.append(
          pl.multiple_of(lo + jnp.minimum(j * _CBLK, span - _CBLK), 8))
    for j in range(nblk):
      k = j % 2
      if j >= 2:
        pltpu.make_async_copy(bufs[k], ox_ref.at[pl.ds(starts[j - 2], _CBLK)],
                              wsems[k]).wait()
      pltpu.make_async_copy(bx_hbm.at[pl.ds(starts[j], _CBLK)], bufs[k],
                            rsems[k]).start()
      pltpu.make_async_copy(bx_hbm.at[pl.ds(starts[j], _CBLK)], bufs[k],
                            rsems[k]).wait()
      pltpu.make_async_copy(bufs[k], ox_ref.at[pl.ds(starts[j], _CBLK)],
                            wsems[k]).start()
    for j in range(max(nblk - 2, 0), nblk):
      k = j % 2
      pltpu.make_async_copy(bufs[k], ox_ref.at[pl.ds(starts[j], _CBLK)],
                            wsems[k]).wait()

  return sc


def _sc_winners(b):
  nchunk = b // _L
  ipt = b // _NW
  cpt = ipt // _L
  mesh = plsc.VectorSubcoreMesh(
      core_axis_name="c", subcore_axis_name="s", num_cores=_NC,
      num_subcores=_NS)

  @functools.partial(
      pl.kernel,
      out_type=jax.ShapeDtypeStruct((b,), jnp.int32),
      mesh=mesh,
      scratch_types=[
          pltpu.VMEM((b,), jnp.int32),
          pltpu.VMEM((ipt,), jnp.int32),
      ],
  )
  def sc(idx_hbm, w_hbm, idx_v, w_v):
    cid = lax.axis_index("c")
    sid = lax.axis_index("s")
    tid = cid * _NS + sid
    base = tid * ipt
    lane = lax.iota(jnp.int32, _L)

    pltpu.sync_copy(idx_hbm, idx_v)
    for j in range(cpt):
      g = tid * cpt + j
      v = idx_v[pl.ds(g * _L, _L)]
      best = g * _L + lane

      def wbody(c, best, v=v):
        u = idx_v[pl.ds(c * _L, _L)]
        for r in range(_L):
          perm = _rot_perm(lane, r)
          ur = u if r == 0 else _gather_lanes(u, perm)
          jr = c * _L + perm
          upd = jnp.logical_and(ur == v, jr > best)
          best = jnp.where(upd, jr, best)
        return best

      w_v[pl.ds(j * _L, _L)] = lax.fori_loop(0, nchunk, wbody, best)
    pltpu.sync_copy(w_v, w_hbm.at[pl.ds(base, ipt)])

  return sc


def _sc_sparse(m, b, d, ox_ref, oy_ref, ot_ref):
  nchunk = b // _L
  ipt = b // _NW
  cpt = ipt // _L
  mesh = plsc.VectorSubcoreMesh(
      core_axis_name="c", subcore_axis_name="s", num_cores=_NC,
      num_subcores=_NS)

  assert cpt == 2

  @functools.partial(
      pl.kernel,
      out_type=(),
      mesh=mesh,
      scratch_types=[
          pltpu.VMEM((cpt, _L), jnp.int32),
          pltpu.VMEM((ipt,), jnp.int32),
          pltpu.VMEM((ipt,), jnp.int32),
          pltpu.VMEM((_L, d), jnp.float32),
          pltpu.VMEM((_L, d), jnp.float32),
          pltpu.VMEM((ipt,), jnp.int32),
          pltpu.VMEM((ipt,), jnp.int32),
          pltpu.VMEM((ipt,), jnp.int32),
      ] + [pltpu.SemaphoreType.DMA] * 9,
  )
  def sc(bx_hbm, inx_hbm, by_hbm, bt_hbm, iny_hbm, idx_hbm, idx3_hbm,
         itv_hbm, w_hbm,
         myidx_v, myflat_v, w_v, rows_a, rows_b, val_v, taily_v,
         tailt_v, s0, s1, s2, s3, s4, s5, s6, s7, s8):
    cid = lax.axis_index("c")
    sid = lax.axis_index("s")
    tid = cid * _NS + sid
    base = tid * ipt
    rows = (rows_a, rows_b)

    pltpu.sync_copy(idx_hbm.at[pl.ds(base, ipt)], myflat_v)
    pltpu.sync_copy(idx3_hbm.at[tid], myidx_v)
    pltpu.sync_copy(w_hbm.at[pl.ds(base, ipt)], w_v)

    tg = [pltpu.async_copy(bx_hbm.at[myidx_v.at[cc]], rows[cc], (s0, s1)[cc])
          for cc in range(cpt)]
    ty = pltpu.async_copy(by_hbm.at[myflat_v], taily_v, s2)
    tt = pltpu.async_copy(bt_hbm.at[myflat_v], tailt_v, s3)

    vg = pltpu.async_copy(iny_hbm.at[w_v], val_v, s8)

    tail_at = pl.ds(pl.multiple_of(m + base, 8), ipt)
    tw = []
    for cc in range(cpt):
      tg[cc].wait()
      tw.append(pltpu.make_async_copy(
          rows[cc],
          ox_ref.at[pl.ds(pl.multiple_of(m + base + cc * _L, 8), _L)],
          (s4, s5)[cc]))
      tw[cc].start()
    ty.wait()
    tyw = pltpu.make_async_copy(taily_v, oy_ref.at[tail_at], s6)
    tyw.start()
    tt.wait()
    ttw = pltpu.make_async_copy(tailt_v, ot_ref.at[tail_at], s7)
    ttw.start()

    vg.wait()
    yw = pltpu.async_copy(val_v, oy_ref.at[myflat_v], s2)
    pltpu.sync_copy(itv_hbm, val_v)
    tvw = pltpu.async_copy(val_v, ot_ref.at[myflat_v], s3)

    sg = []
    for cc in range(cpt):
      tw[cc].wait()
      sg.append(pltpu.async_copy(inx_hbm.at[w_v.at[pl.ds(cc * _L, _L)]],
                                 rows[cc], (s0, s1)[cc]))
    sw = []
    for cc in range(cpt):
      sg[cc].wait()
      sw.append(pltpu.async_copy(rows[cc], ox_ref.at[myidx_v.at[cc]],
                                 (s4, s5)[cc]))
    for c in sw:
      c.wait()
    yw.wait()
    tvw.wait()
    tyw.wait()
    ttw.wait()

  return sc


def kernel(bx, by, bt, in_x, in_y, in_t, swap_idx):
  m = bx.shape[0]
  b = in_x.shape[0]
  d = 1
  for s in bx.shape[1:]:
    d *= s
  mp = ((m + 127) // 128) * 128
  assert b % (_NW * _L) == 0 and m % _BLK == 0 and mp <= m + b

  bx_f = bx.reshape(m, d)
  inx_f = in_x.reshape(b, d)
  idx3 = swap_idx.reshape(_NW, b // _NW // _L, _L)
  itv = jnp.full((b // _NW,), in_t, dtype=jnp.int32)
  pad = jnp.zeros((mp - m,), dtype=jnp.int32)
  by_p = jnp.concatenate([by, pad])
  bt_p = jnp.concatenate([bt, pad])

  w = _sc_winners(b)(swap_idx)
  body_x, body_y, body_t = _tc_body_copy(m, mp, b, d)(bx_f, by_p, bt_p)
  ox_ref = jax.new_ref(body_x)
  oy_ref = jax.new_ref(body_y)
  ot_ref = jax.new_ref(body_t)
  _sc_sparse(m, b, d, ox_ref, oy_ref, ot_ref)(
      bx_f, inx_f, by, bt, in_y, swap_idx, idx3, itv, w)
  out_bx = ox_ref[...].reshape((m + b,) + bx.shape[1:])
  return (out_bx, oy_ref[...], ot_ref[...])


# speedup vs baseline: 1.0297x; 1.0005x over previous
import functools

import jax
import jax.numpy as jnp
from jax import lax
from jax.experimental import pallas as pl
from jax.experimental.pallas import tpu as pltpu
from jax.experimental.pallas import tpu_sc as plsc

_NC = 2
_NS = 16
_NW = _NC * _NS
_L = 16
_BLK = 1000


def _rot_perm(lane, r):
  return (lane + r) & (_L - 1)


def _gather_lanes(x, perm):
  return jnp.take_along_axis(x, perm, axis=0,
                             mode=lax.GatherScatterMode.PROMISE_IN_BOUNDS)


def _tc_body_copy(m, mp, b, d):
  def body(bx_ref, by_ref, bt_ref, ox_ref, oy_ref, ot_ref, sem):
    i = pl.program_id(0)
    ox_ref[...] = bx_ref[...]

    @pl.when(i == 0)
    def _ints():
      pltpu.make_async_copy(by_ref, oy_ref.at[pl.ds(0, mp)], sem).start()
      pltpu.make_async_copy(bt_ref, ot_ref.at[pl.ds(0, mp)], sem).start()
      pltpu.make_async_copy(by_ref, oy_ref.at[pl.ds(0, mp)], sem).wait()
      pltpu.make_async_copy(bt_ref, ot_ref.at[pl.ds(0, mp)], sem).wait()

  return pl.pallas_call(
      body,
      grid=(m // _BLK,),
      in_specs=[pl.BlockSpec((_BLK, d), lambda i: (i, 0)),
                pl.BlockSpec(memory_space=pl.ANY),
                pl.BlockSpec(memory_space=pl.ANY)],
      out_specs=[pl.BlockSpec((_BLK, d), lambda i: (i, 0)),
                 pl.BlockSpec(memory_space=pl.ANY),
                 pl.BlockSpec(memory_space=pl.ANY)],
      out_shape=(jax.ShapeDtypeStruct((m + b, d), jnp.float32),
                 jax.ShapeDtypeStruct((m + b,), jnp.int32),
                 jax.ShapeDtypeStruct((m + b,), jnp.int32)),
      scratch_shapes=[pltpu.SemaphoreType.DMA],
  )


def _sc_winners(b):
  nchunk = b // _L
  ipt = b // _NW
  cpt = ipt // _L
  mesh = plsc.VectorSubcoreMesh(
      core_axis_name="c", subcore_axis_name="s", num_cores=_NC,
      num_subcores=_NS)

  @functools.partial(
      pl.kernel,
      out_type=jax.ShapeDtypeStruct((b,), jnp.int32),
      mesh=mesh,
      scratch_types=[
          pltpu.VMEM((b,), jnp.int32),
          pltpu.VMEM((ipt,), jnp.int32),
      ],
  )
  def sc(idx_hbm, w_hbm, idx_v, w_v):
    cid = lax.axis_index("c")
    sid = lax.axis_index("s")
    tid = cid * _NS + sid
    base = tid * ipt
    lane = lax.iota(jnp.int32, _L)

    pltpu.sync_copy(idx_hbm, idx_v)
    for j in range(cpt):
      g = tid * cpt + j
      v = idx_v[pl.ds(g * _L, _L)]
      best = g * _L + lane

      def wbody(c, best, v=v):
        u = idx_v[pl.ds(c * _L, _L)]
        for r in range(_L):
          perm = _rot_perm(lane, r)
          ur = u if r == 0 else _gather_lanes(u, perm)
          jr = c * _L + perm
          upd = jnp.logical_and(ur == v, jr > best)
          best = jnp.where(upd, jr, best)
        return best

      w_v[pl.ds(j * _L, _L)] = lax.fori_loop(0, nchunk, wbody, best)
    pltpu.sync_copy(w_v, w_hbm.at[pl.ds(base, ipt)])

  return sc


def _sc_sparse(m, b, d, ox_ref, oy_ref, ot_ref):
  nchunk = b // _L
  ipt = b // _NW
  cpt = ipt // _L
  mesh = plsc.VectorSubcoreMesh(
      core_axis_name="c", subcore_axis_name="s", num_cores=_NC,
      num_subcores=_NS)

  nch = 4
  rw = ipt // nch
  assert rw % 8 == 0 or rw == 8

  @functools.partial(
      pl.kernel,
      out_type=(),
      mesh=mesh,
      scratch_types=[
          pltpu.VMEM((nch, rw), jnp.int32),
          pltpu.VMEM((ipt,), jnp.int32),
          pltpu.VMEM((ipt,), jnp.int32),
          pltpu.VMEM((rw, d), jnp.float32),
          pltpu.VMEM((rw, d), jnp.float32),
          pltpu.VMEM((rw, d), jnp.float32),
          pltpu.VMEM((rw, d), jnp.float32),
          pltpu.VMEM((ipt,), jnp.int32),
          pltpu.VMEM((ipt,), jnp.int32),
          pltpu.VMEM((ipt,), jnp.int32),
          pltpu.VMEM((ipt,), jnp.int32),
      ] + [pltpu.SemaphoreType.DMA] * 12,
  )
  def sc(bx_hbm, inx_hbm, by_hbm, bt_hbm, iny_hbm, idx_hbm, idx3_hbm,
         itv_hbm, w_hbm,
         myidx_v, myflat_v, w_v, ta, tb, ra, rb, val_v, itv_v, taily_v,
         tailt_v, tg0, tg1, tw0, tw1, sg0, sg1, sw0, sw1, iy, it, iv, ix):
    cid = lax.axis_index("c")
    sid = lax.axis_index("s")
    tid = cid * _NS + sid
    base = tid * ipt
    tbuf, rbuf = (ta, tb), (ra, rb)
    tgs, tws = (tg0, tg1), (tw0, tw1)
    sgs, sws = (sg0, sg1), (sw0, sw1)

    pltpu.sync_copy(idx_hbm.at[pl.ds(base, ipt)], myflat_v)
    pltpu.sync_copy(idx3_hbm.at[tid], myidx_v)
    pltpu.sync_copy(w_hbm.at[pl.ds(base, ipt)], w_v)

    ty = pltpu.async_copy(by_hbm.at[myflat_v], taily_v, iy)
    tt = pltpu.async_copy(bt_hbm.at[myflat_v], tailt_v, it)
    vg = pltpu.async_copy(iny_hbm.at[w_v], val_v, iv)
    iw = pltpu.async_copy(itv_hbm, itv_v, ix)

    tg = [None] * nch
    tw = [None] * nch
    sg = [None] * nch
    sw = [None] * nch
    for cc in range(nch):
      k = cc % 2
      if cc >= 2:
        tw[cc - 2].wait()
        sw[cc - 2].wait()
      tg[cc] = pltpu.async_copy(bx_hbm.at[myidx_v.at[cc]], tbuf[k], tgs[k])
      sg[cc] = pltpu.async_copy(inx_hbm.at[w_v.at[pl.ds(cc * rw, rw)]],
                                rbuf[k], sgs[k])
      tg[cc].wait()
      tw[cc] = pltpu.make_async_copy(
          tbuf[k],
          ox_ref.at[pl.ds(pl.multiple_of(m + base + cc * rw, 8), rw)],
          tws[k])
      tw[cc].start()
      sg[cc].wait()
      sw[cc] = pltpu.async_copy(rbuf[k], ox_ref.at[myidx_v.at[cc]], sws[k])

    tail_at = pl.ds(pl.multiple_of(m + base, 8), ipt)
    ty.wait()
    tyw = pltpu.make_async_copy(taily_v, oy_ref.at[tail_at], iy)
    tyw.start()
    tt.wait()
    ttw = pltpu.make_async_copy(tailt_v, ot_ref.at[tail_at], it)
    ttw.start()
    vg.wait()
    yw = pltpu.async_copy(val_v, oy_ref.at[myflat_v], iv)
    iw.wait()
    tvw = pltpu.async_copy(itv_v, ot_ref.at[myflat_v], ix)

    for c in (tw[2], tw[3], sw[2], sw[3], tyw, ttw, yw, tvw):
      c.wait()

  return sc


def kernel(bx, by, bt, in_x, in_y, in_t, swap_idx):
  m = bx.shape[0]
  b = in_x.shape[0]
  d = 1
  for s in bx.shape[1:]:
    d *= s
  mp = ((m + 127) // 128) * 128
  assert b % (_NW * _L) == 0 and m % _BLK == 0 and mp <= m + b

  bx_f = bx.reshape(m, d)
  inx_f = in_x.reshape(b, d)
  idx3 = swap_idx.reshape(_NW, 4, b // _NW // 4)
  itv = jnp.full((b // _NW,), in_t, dtype=jnp.int32)
  pad = jnp.zeros((mp - m,), dtype=jnp.int32)
  by_p = jnp.concatenate([by, pad])
  bt_p = jnp.concatenate([bt, pad])

  w = _sc_winners(b)(swap_idx)
  body_x, body_y, body_t = _tc_body_copy(m, mp, b, d)(bx_f, by_p, bt_p)
  ox_ref = jax.new_ref(body_x)
  oy_ref = jax.new_ref(body_y)
  ot_ref = jax.new_ref(body_t)
  _sc_sparse(m, b, d, ox_ref, oy_ref, ot_ref)(
      bx_f, inx_f, by, bt, in_y, swap_idx, idx3, itv, w)
  out_bx = ox_ref[...].reshape((m + b,) + bx.shape[1:])
  return (out_bx, oy_ref[...], ot_ref[...])
